# baseline (speedup 1.0000x reference)
import jax
import jax.numpy as jnp
from jax.experimental import pallas as pl

_N = 4096
_BR = 128
_CHUNK = 256
_NT = _N // 128


def _bucket_body(x_ref, bkt_ref):
    x = x_ref[...]
    u = jax.lax.bitcast_convert_type(x, jnp.uint32)
    top = jnp.uint32(0x80000000)
    s = jnp.where(u >= top, ~u, u | top)

    def body(t, carry):
        p1, p2 = carry
        bit = jnp.uint32(1) << (31 - t).astype(jnp.uint32)
        c1 = p1 | bit
        c2 = p2 | bit
        n1 = jnp.sum((s < c1).astype(jnp.int32))
        n2 = jnp.sum((s < c2).astype(jnp.int32))
        p1 = jnp.where(n1 <= 1365, c1, p1)
        p2 = jnp.where(n2 <= 2730, c2, p2)
        return p1, p2

    p1, p2 = jax.lax.fori_loop(0, 32, body,
                               (jnp.uint32(0), jnp.uint32(0)))
    t1 = jax.lax.bitcast_convert_type(
        jnp.where(p1 >= top, p1 ^ top, ~p1), jnp.float32)
    t2 = jax.lax.bitcast_convert_type(
        jnp.where(p2 >= top, p2 ^ top, ~p2), jnp.float32)
    bkt_ref[...] = ((x > t1).astype(jnp.int32)
                    + (x > t2).astype(jnp.int32))


def _graph_body(ir_ref, br_ref, ic_ref, bc_ref, rl_ref, m_ref,
                adj_ref, p_ref):
    sa = ir_ref[...] == ic_ref[...]
    sb = br_ref[...] == bc_ref[...]
    adj_ref[...] = jnp.where(sa, 1.0, jnp.where(sb, 0.2, 0.0)
                             ).astype(jnp.float32)
    p_ref[...] = (rl_ref[...] == m_ref[...]).astype(jnp.float32)

    i = pl.program_id(0)
    r0 = i * _BR
    rows = jax.lax.broadcasted_iota(jnp.int32, (_BR, _BR), 0)
    cols = jax.lax.broadcasted_iota(jnp.int32, (_BR, _BR), 1)
    dmask = (rows != cols).astype(jnp.float32)
    adj_ref[:, pl.ds(r0, _BR)] = adj_ref[:, pl.ds(r0, _BR)] * dmask
    rows3 = jax.lax.broadcasted_iota(jnp.int32, (_BR, 2, 128), 0)
    cols3 = jax.lax.broadcasted_iota(jnp.int32, (_BR, 2, 128), 2)
    dmask3 = (rows3 != cols3).astype(jnp.float32)
    p_ref[:, pl.ds(2 * i, 2), :] = p_ref[:, pl.ds(2 * i, 2), :] * dmask3


def kernel(industry, x_style):
    n = _N
    ind = industry.astype(jnp.int32)
    x = x_style[:, 0]

    bkt2d = pl.pallas_call(
        _bucket_body,
        out_shape=jax.ShapeDtypeStruct((32, 128), jnp.int32),
    )(x.reshape(32, 128))
    bkt = bkt2d.reshape(n)

    l0 = ind * 2
    l1 = bkt * 2 + 1
    m = jnp.stack([l0.reshape(_NT, 128), l1.reshape(_NT, 128)],
                  axis=1).reshape(1, 2 * _NT, 128)
    rl = jnp.broadcast_to(jnp.stack([l0, l1], axis=1)[:, None, :],
                          (n, _NT, 2)).reshape(n, 2 * _NT, 1)

    nblk = n // _BR
    adj, p = pl.pallas_call(
        _graph_body,
        grid=(nblk,),
        in_specs=[
            pl.BlockSpec((_BR, 1), lambda i: (i, 0)),
            pl.BlockSpec((_BR, 1), lambda i: (i, 0)),
            pl.BlockSpec((1, n), lambda i: (0, 0)),
            pl.BlockSpec((1, n), lambda i: (0, 0)),
            pl.BlockSpec((_BR, 2 * _NT, 1), lambda i: (i, 0, 0)),
            pl.BlockSpec((1, 2 * _NT, 128), lambda i: (0, 0, 0)),
        ],
        out_specs=[
            pl.BlockSpec((_BR, n), lambda i: (i, 0)),
            pl.BlockSpec((_BR, 2 * _NT, 128), lambda i: (i, 0, 0)),
        ],
        out_shape=[
            jax.ShapeDtypeStruct((n, n), jnp.float32),
            jax.ShapeDtypeStruct((n, 2 * _NT, 128), jnp.float32),
        ],
    )(ind.reshape(n, 1), bkt.reshape(n, 1),
      ind.reshape(1, n), bkt.reshape(1, n), rl, m)

    feat = jnp.transpose(p.reshape(n, _NT, 2, 128),
                         (0, 1, 3, 2)).reshape(n, n, 2)
    return adj, feat


# speedup vs baseline: 1.0148x; 1.0148x over previous
import jax
import jax.numpy as jnp
from jax.experimental import pallas as pl

_N = 4096
_BR = 256
_CHUNK = 256
_NT = _N // 128


def _bucket_body(x_ref, bkt_ref):
    x = x_ref[...]
    u = jax.lax.bitcast_convert_type(x, jnp.uint32)
    top = jnp.uint32(0x80000000)
    s = jnp.where(u >= top, ~u, u | top)

    def body(t, carry):
        p1, p2 = carry
        bit = jnp.uint32(1) << (31 - t).astype(jnp.uint32)
        c1 = p1 | bit
        c2 = p2 | bit
        n1 = jnp.sum((s < c1).astype(jnp.int32))
        n2 = jnp.sum((s < c2).astype(jnp.int32))
        p1 = jnp.where(n1 <= 1365, c1, p1)
        p2 = jnp.where(n2 <= 2730, c2, p2)
        return p1, p2

    p1, p2 = jax.lax.fori_loop(0, 32, body,
                               (jnp.uint32(0), jnp.uint32(0)))
    t1 = jax.lax.bitcast_convert_type(
        jnp.where(p1 >= top, p1 ^ top, ~p1), jnp.float32)
    t2 = jax.lax.bitcast_convert_type(
        jnp.where(p2 >= top, p2 ^ top, ~p2), jnp.float32)
    bkt_ref[...] = ((x > t1).astype(jnp.int32)
                    + (x > t2).astype(jnp.int32))


def _graph_body(ir_ref, br_ref, ic_ref, bc_ref, rl_ref, m_ref,
                adj_ref, p_ref):
    sa = ir_ref[...] == ic_ref[...]
    sb = br_ref[...] == bc_ref[...]
    adj_ref[...] = jnp.where(sa, 1.0, jnp.where(sb, 0.2, 0.0)
                             ).astype(jnp.float32)
    p_ref[...] = (rl_ref[...] == m_ref[...]).astype(jnp.float32)

    i = pl.program_id(0)
    r0 = i * _BR
    rows = jax.lax.broadcasted_iota(jnp.int32, (_BR, _BR), 0)
    cols = jax.lax.broadcasted_iota(jnp.int32, (_BR, _BR), 1)
    dmask = (rows != cols).astype(jnp.float32)
    adj_ref[:, pl.ds(r0, _BR)] = adj_ref[:, pl.ds(r0, _BR)] * dmask
    nj = _BR // 128
    rows3 = jax.lax.broadcasted_iota(jnp.int32, (_BR, 2 * nj, 128), 0)
    rr3 = jax.lax.broadcasted_iota(jnp.int32, (_BR, 2 * nj, 128), 1)
    cols3 = jax.lax.broadcasted_iota(jnp.int32, (_BR, 2 * nj, 128), 2)
    ondiag = ((rows3 // 128) == (rr3 // 2)) & (cols3 == (rows3 % 128))
    dmask3 = 1.0 - ondiag.astype(jnp.float32)
    p_ref[:, pl.ds(2 * nj * i, 2 * nj), :] = (
        p_ref[:, pl.ds(2 * nj * i, 2 * nj), :] * dmask3)


def kernel(industry, x_style):
    n = _N
    ind = industry.astype(jnp.int32)
    x = x_style[:, 0]

    bkt2d = pl.pallas_call(
        _bucket_body,
        out_shape=jax.ShapeDtypeStruct((32, 128), jnp.int32),
    )(x.reshape(32, 128))
    bkt = bkt2d.reshape(n)

    l0 = ind * 2
    l1 = bkt * 2 + 1
    m = jnp.stack([l0.reshape(_NT, 128), l1.reshape(_NT, 128)],
                  axis=1).reshape(1, 2 * _NT, 128)
    rl = jnp.broadcast_to(jnp.stack([l0, l1], axis=1)[:, None, :],
                          (n, _NT, 2)).reshape(n, 2 * _NT, 1)

    nblk = n // _BR
    adj, p = pl.pallas_call(
        _graph_body,
        grid=(nblk,),
        in_specs=[
            pl.BlockSpec((_BR, 1), lambda i: (i, 0)),
            pl.BlockSpec((_BR, 1), lambda i: (i, 0)),
            pl.BlockSpec((1, n), lambda i: (0, 0)),
            pl.BlockSpec((1, n), lambda i: (0, 0)),
            pl.BlockSpec((_BR, 2 * _NT, 1), lambda i: (i, 0, 0)),
            pl.BlockSpec((1, 2 * _NT, 128), lambda i: (0, 0, 0)),
        ],
        out_specs=[
            pl.BlockSpec((_BR, n), lambda i: (i, 0)),
            pl.BlockSpec((_BR, 2 * _NT, 128), lambda i: (i, 0, 0)),
        ],
        out_shape=[
            jax.ShapeDtypeStruct((n, n), jnp.float32),
            jax.ShapeDtypeStruct((n, 2 * _NT, 128), jnp.float32),
        ],
    )(ind.reshape(n, 1), bkt.reshape(n, 1),
      ind.reshape(1, n), bkt.reshape(1, n), rl, m)

    feat = jnp.transpose(p.reshape(n, _NT, 2, 128),
                         (0, 1, 3, 2)).reshape(n, n, 2)
    return adj, feat
